# Initial kernel scaffold; baseline (speedup 1.0000x reference)
import jax
import jax.numpy as jnp
from jax.experimental import pallas as pl
from jax.experimental.pallas import tpu as pltpu

H = W = 100
CIN, C1, C2 = 3, 10, 20
K = 5
OH1 = OW1 = H - K + 1
PH1 = PW1 = OH1 // 2
OH2 = OW2 = PH1 - K + 1
PH2 = PW2 = OH2 // 2
NFEAT = C2 * PH2 * PW2
H1, NCLS = 50, 3
B_TILE = 16
B_HEAD = 128


def _conv_kernel(x_ref, r1e_ref, r1o_ref, b1_ref, r2e_ref, r2o_ref, b2_ref,
                 a2_ref, lhs1_ref, a1_ref, lhs2_ref):
    f32 = jnp.float32
    bt = x_ref.shape[0]

    for kh in range(K):
        lhs1_ref[:, :, kh * CIN * W:(kh + 1) * CIN * W] = x_ref[:, kh:kh + OH1, :]
    lhs1 = lhs1_ref[...].reshape(bt * OH1, CIN * K * W)
    ye = jnp.dot(lhs1, r1e_ref[...], preferred_element_type=f32)
    yo = jnp.dot(lhs1, r1o_ref[...], preferred_element_type=f32)
    m = (jnp.maximum(ye, yo) + b1_ref[...]).reshape(bt, OH1, C1 * PW1)
    a1_ref[...] = jnp.maximum(
        jnp.maximum(m[:, ::2, :], m[:, 1::2, :]), 0.0).astype(jnp.bfloat16)

    for kh in range(K):
        lhs2_ref[:, :, kh * C1 * PW1:(kh + 1) * C1 * PW1] = a1_ref[:, kh:kh + OH2, :]
    lhs2 = lhs2_ref[...].reshape(bt * OH2, C1 * K * PW1)
    ye2 = jnp.dot(lhs2, r2e_ref[...], preferred_element_type=f32)
    yo2 = jnp.dot(lhs2, r2o_ref[...], preferred_element_type=f32)
    m2 = (jnp.maximum(ye2, yo2) + b2_ref[...]).reshape(bt, OH2, C2 * PW2)
    a2_ref[...] = jnp.maximum(
        jnp.maximum(m2[:, ::2, :], m2[:, 1::2, :]), 0.0).astype(jnp.bfloat16)


def _head_kernel(f_ref, w1_ref, b1_ref, w2_ref, b2_ref, o_ref):
    f32 = jnp.float32
    h = jnp.dot(f_ref[...], w1_ref[...], preferred_element_type=f32) + b1_ref[...]
    h = jnp.maximum(h, 0.0)
    o_ref[...] = jnp.dot(h, w2_ref[...], preferred_element_type=f32) + b2_ref[...]


def _banded(w, win, wout):
    wcol = jnp.arange(win)[:, None]
    ow = jnp.arange(wout)[None, :]
    kw = wcol - ow
    mask = (kw >= 0) & (kw < K)
    kwc = jnp.clip(kw, 0, K - 1)
    band = jnp.where(mask[None, None, None], w[:, :, :, kwc], 0.0)
    cout, cin = w.shape[0], w.shape[1]
    return jnp.transpose(band, (2, 1, 3, 0, 4)).reshape(K * cin * win, cout * wout)


def _even_odd(r, cout, wout):
    r4 = r.reshape(r.shape[0], cout, wout // 2, 2)
    bf16 = jnp.bfloat16
    return (r4[..., 0].reshape(r.shape[0], cout * (wout // 2)).astype(bf16),
            r4[..., 1].reshape(r.shape[0], cout * (wout // 2)).astype(bf16))


def kernel(x, conv1_w, conv1_b, conv2_w, conv2_b, fc1_w, fc1_b, fc2_w, fc2_b):
    n = x.shape[0]
    bt = min(B_TILE, n)
    num_tiles = -(-n // bt)
    n_pad = num_tiles * bt

    xr = jnp.transpose(x, (0, 2, 1, 3)).reshape(n, H, CIN * W).astype(jnp.bfloat16)
    if n_pad != n:
        xr = jnp.concatenate(
            [xr, jnp.zeros((n_pad - n,) + xr.shape[1:], xr.dtype)], axis=0)

    r1e, r1o = _even_odd(_banded(conv1_w, W, OW1), C1, OW1)
    r2e, r2o = _even_odd(_banded(conv2_w, PW1, OW2), C2, OW2)
    b1r = jnp.repeat(conv1_b, PW1).reshape(1, C1 * PW1)
    b2r = jnp.repeat(conv2_b, PW2).reshape(1, C2 * PW2)
    fw1 = jnp.transpose(fc1_w.reshape(H1, C2, PH2, PW2),
                        (2, 1, 3, 0)).reshape(NFEAT, H1).astype(jnp.bfloat16)
    fb1 = fc1_b.reshape(1, H1)
    fw2 = fc2_w.T
    fb2 = fc2_b.reshape(1, NCLS)

    def full(shape):
        zeros = (0,) * len(shape)
        return pl.BlockSpec(shape, lambda g: zeros)

    a2 = pl.pallas_call(
        _conv_kernel,
        out_shape=jax.ShapeDtypeStruct((n_pad, PH2, C2 * PW2), jnp.bfloat16),
        grid=(num_tiles,),
        in_specs=[
            pl.BlockSpec((bt, H, CIN * W), lambda g: (g, 0, 0)),
            full((CIN * K * W, C1 * PW1)), full((CIN * K * W, C1 * PW1)),
            full((1, C1 * PW1)),
            full((C1 * K * PW1, C2 * PW2)), full((C1 * K * PW1, C2 * PW2)),
            full((1, C2 * PW2)),
        ],
        out_specs=pl.BlockSpec((bt, PH2, C2 * PW2), lambda g: (g, 0, 0)),
        scratch_shapes=[
            pltpu.VMEM((bt, OH1, CIN * K * W), jnp.bfloat16),
            pltpu.VMEM((bt, PH1, C1 * PW1), jnp.bfloat16),
            pltpu.VMEM((bt, OH2, C1 * K * PW1), jnp.bfloat16),
        ],
        compiler_params=pltpu.CompilerParams(
            dimension_semantics=("parallel",),
            vmem_limit_bytes=56 * 1024 * 1024),
    )(xr, r1e, r1o, b1r, r2e, r2o, b2r)

    feat = a2.reshape(n_pad, NFEAT)
    bh = min(B_HEAD, n_pad)
    hv_tiles = -(-n_pad // bh)
    n_pad2 = hv_tiles * bh
    if n_pad2 != n_pad:
        feat = jnp.concatenate(
            [feat, jnp.zeros((n_pad2 - n_pad, NFEAT), feat.dtype)], axis=0)

    out = pl.pallas_call(
        _head_kernel,
        out_shape=jax.ShapeDtypeStruct((n_pad2, NCLS), jnp.float32),
        grid=(hv_tiles,),
        in_specs=[
            pl.BlockSpec((bh, NFEAT), lambda g: (g, 0)),
            full((NFEAT, H1)), full((1, H1)),
            full((H1, NCLS)), full((1, NCLS)),
        ],
        out_specs=pl.BlockSpec((bh, NCLS), lambda g: (g, 0)),
        compiler_params=pltpu.CompilerParams(
            dimension_semantics=("parallel",),
            vmem_limit_bytes=56 * 1024 * 1024),
    )(feat, fw1, fb1, fw2, fb2)
    return out[:n]


# speedup vs baseline: 1.6535x; 1.6535x over previous
import jax
import jax.numpy as jnp
from jax.experimental import pallas as pl
from jax.experimental.pallas import tpu as pltpu

H = W = 100
CIN, C1, C2 = 3, 10, 20
K = 5
OH1 = OW1 = H - K + 1
PH1 = PW1 = OH1 // 2
OH2 = OW2 = PH1 - K + 1
PH2 = PW2 = OH2 // 2
NFEAT = C2 * PH2 * PW2
H1, NCLS = 50, 3
B_TILE = 16
B_HEAD = 128


def _conv_kernel(x_ref, r1e_ref, r1o_ref, b1_ref, r2e_ref, r2o_ref, b2_ref,
                 a2_ref, lhs1_ref, a1_ref, lhs2_ref):
    f32 = jnp.float32
    bt = x_ref.shape[0]

    for kh in range(K):
        lhs1_ref[:, :, kh * CIN * W:(kh + 1) * CIN * W] = x_ref[:, kh:kh + OH1, :]
    lhs1 = lhs1_ref[...].reshape(bt * OH1, CIN * K * W)
    ye = jnp.dot(lhs1, r1e_ref[...], preferred_element_type=f32)
    yo = jnp.dot(lhs1, r1o_ref[...], preferred_element_type=f32)
    m = (jnp.maximum(ye, yo) + b1_ref[...]).reshape(bt, PH1, 2, C1 * PW1)
    a1_ref[...] = jnp.maximum(
        jnp.maximum(m[:, :, 0, :], m[:, :, 1, :]), 0.0).astype(jnp.bfloat16)

    for kh in range(K):
        lhs2_ref[:, :, kh * C1 * PW1:(kh + 1) * C1 * PW1] = a1_ref[:, kh:kh + OH2, :]
    lhs2 = lhs2_ref[...].reshape(bt * OH2, C1 * K * PW1)
    ye2 = jnp.dot(lhs2, r2e_ref[...], preferred_element_type=f32)
    yo2 = jnp.dot(lhs2, r2o_ref[...], preferred_element_type=f32)
    m2 = (jnp.maximum(ye2, yo2) + b2_ref[...]).reshape(bt, PH2, 2, C2 * PW2)
    a2_ref[...] = jnp.maximum(
        jnp.maximum(m2[:, :, 0, :], m2[:, :, 1, :]), 0.0).astype(jnp.bfloat16)


def _head_kernel(f_ref, w1_ref, b1_ref, w2_ref, b2_ref, o_ref):
    f32 = jnp.float32
    h = jnp.dot(f_ref[...], w1_ref[...], preferred_element_type=f32) + b1_ref[...]
    h = jnp.maximum(h, 0.0)
    o_ref[...] = jnp.dot(h, w2_ref[...], preferred_element_type=f32) + b2_ref[...]


def _banded(w, win, wout):
    wcol = jnp.arange(win)[:, None]
    ow = jnp.arange(wout)[None, :]
    kw = wcol - ow
    mask = (kw >= 0) & (kw < K)
    kwc = jnp.clip(kw, 0, K - 1)
    band = jnp.where(mask[None, None, None], w[:, :, :, kwc], 0.0)
    cout, cin = w.shape[0], w.shape[1]
    return jnp.transpose(band, (2, 1, 3, 0, 4)).reshape(K * cin * win, cout * wout)


def _even_odd(r, cout, wout):
    r4 = r.reshape(r.shape[0], cout, wout // 2, 2)
    bf16 = jnp.bfloat16
    return (r4[..., 0].reshape(r.shape[0], cout * (wout // 2)).astype(bf16),
            r4[..., 1].reshape(r.shape[0], cout * (wout // 2)).astype(bf16))


def kernel(x, conv1_w, conv1_b, conv2_w, conv2_b, fc1_w, fc1_b, fc2_w, fc2_b):
    n = x.shape[0]
    bt = min(B_TILE, n)
    num_tiles = -(-n // bt)
    n_pad = num_tiles * bt

    xr = jnp.transpose(x, (0, 2, 1, 3)).reshape(n, H, CIN * W).astype(jnp.bfloat16)
    if n_pad != n:
        xr = jnp.concatenate(
            [xr, jnp.zeros((n_pad - n,) + xr.shape[1:], xr.dtype)], axis=0)

    r1e, r1o = _even_odd(_banded(conv1_w, W, OW1), C1, OW1)
    r2e, r2o = _even_odd(_banded(conv2_w, PW1, OW2), C2, OW2)
    b1r = jnp.repeat(conv1_b, PW1).reshape(1, C1 * PW1)
    b2r = jnp.repeat(conv2_b, PW2).reshape(1, C2 * PW2)
    fw1 = jnp.transpose(fc1_w.reshape(H1, C2, PH2, PW2),
                        (2, 1, 3, 0)).reshape(NFEAT, H1).astype(jnp.bfloat16)
    fb1 = fc1_b.reshape(1, H1)
    fw2 = fc2_w.T
    fb2 = fc2_b.reshape(1, NCLS)

    def full(shape):
        zeros = (0,) * len(shape)
        return pl.BlockSpec(shape, lambda g: zeros)

    a2 = pl.pallas_call(
        _conv_kernel,
        out_shape=jax.ShapeDtypeStruct((n_pad, PH2, C2 * PW2), jnp.bfloat16),
        grid=(num_tiles,),
        in_specs=[
            pl.BlockSpec((bt, H, CIN * W), lambda g: (g, 0, 0)),
            full((CIN * K * W, C1 * PW1)), full((CIN * K * W, C1 * PW1)),
            full((1, C1 * PW1)),
            full((C1 * K * PW1, C2 * PW2)), full((C1 * K * PW1, C2 * PW2)),
            full((1, C2 * PW2)),
        ],
        out_specs=pl.BlockSpec((bt, PH2, C2 * PW2), lambda g: (g, 0, 0)),
        scratch_shapes=[
            pltpu.VMEM((bt, OH1, CIN * K * W), jnp.bfloat16),
            pltpu.VMEM((bt, PH1, C1 * PW1), jnp.bfloat16),
            pltpu.VMEM((bt, OH2, C1 * K * PW1), jnp.bfloat16),
        ],
        compiler_params=pltpu.CompilerParams(
            dimension_semantics=("parallel",),
            vmem_limit_bytes=56 * 1024 * 1024),
    )(xr, r1e, r1o, b1r, r2e, r2o, b2r)

    feat = a2.reshape(n_pad, NFEAT)
    bh = min(B_HEAD, n_pad)
    hv_tiles = -(-n_pad // bh)
    n_pad2 = hv_tiles * bh
    if n_pad2 != n_pad:
        feat = jnp.concatenate(
            [feat, jnp.zeros((n_pad2 - n_pad, NFEAT), feat.dtype)], axis=0)

    out = pl.pallas_call(
        _head_kernel,
        out_shape=jax.ShapeDtypeStruct((n_pad2, NCLS), jnp.float32),
        grid=(hv_tiles,),
        in_specs=[
            pl.BlockSpec((bh, NFEAT), lambda g: (g, 0)),
            full((NFEAT, H1)), full((1, H1)),
            full((H1, NCLS)), full((1, NCLS)),
        ],
        out_specs=pl.BlockSpec((bh, NCLS), lambda g: (g, 0)),
        compiler_params=pltpu.CompilerParams(
            dimension_semantics=("parallel",),
            vmem_limit_bytes=56 * 1024 * 1024),
    )(feat, fw1, fb1, fw2, fb2)
    return out[:n]


# speedup vs baseline: 2.9915x; 1.8092x over previous
import jax
import jax.numpy as jnp
from jax.experimental import pallas as pl
from jax.experimental.pallas import tpu as pltpu

H = W = 100
CIN, C1, C2 = 3, 10, 20
K = 5
OH1 = OW1 = H - K + 1
PH1 = PW1 = OH1 // 2
OH2 = OW2 = PH1 - K + 1
PH2 = PW2 = OH2 // 2
NFEAT = C2 * PH2 * PW2
H1, NCLS = 50, 3
B_TILE = 32
B_HEAD = 128


def _conv_kernel(x_ref, r1e_ref, r1o_ref, b1_ref, r2e_ref, r2o_ref, b2_ref,
                 a2_ref, lhs1_ref, a1_ref, lhs2_ref):
    f32 = jnp.float32
    bt = x_ref.shape[0]

    for kh in range(K):
        lhs1_ref[:, :, kh * CIN * W:(kh + 1) * CIN * W] = x_ref[:, kh:kh + OH1, :]
    lhs1 = lhs1_ref[...].reshape(bt * OH1, CIN * K * W)
    ye = jnp.dot(lhs1, r1e_ref[...], preferred_element_type=f32)
    yo = jnp.dot(lhs1, r1o_ref[...], preferred_element_type=f32)
    m = (jnp.maximum(ye, yo) + b1_ref[...]).reshape(bt, PH1, 2, C1 * PW1)
    a1_ref[...] = jnp.maximum(
        jnp.maximum(m[:, :, 0, :], m[:, :, 1, :]), 0.0).astype(jnp.bfloat16)

    for kh in range(K):
        lhs2_ref[:, :, kh * C1 * PW1:(kh + 1) * C1 * PW1] = a1_ref[:, kh:kh + OH2, :]
    lhs2 = lhs2_ref[...].reshape(bt * OH2, C1 * K * PW1)
    ye2 = jnp.dot(lhs2, r2e_ref[...], preferred_element_type=f32)
    yo2 = jnp.dot(lhs2, r2o_ref[...], preferred_element_type=f32)
    m2 = (jnp.maximum(ye2, yo2) + b2_ref[...]).reshape(bt, PH2, 2, C2 * PW2)
    a2_ref[...] = jnp.maximum(
        jnp.maximum(m2[:, :, 0, :], m2[:, :, 1, :]), 0.0).astype(jnp.bfloat16)


def _head_kernel(f_ref, w1_ref, b1_ref, w2_ref, b2_ref, o_ref):
    f32 = jnp.float32
    h = jnp.dot(f_ref[...], w1_ref[...], preferred_element_type=f32) + b1_ref[...]
    h = jnp.maximum(h, 0.0)
    o_ref[...] = jnp.dot(h, w2_ref[...], preferred_element_type=f32) + b2_ref[...]


def _banded_eo(w, win, wout):
    k_ = jnp.arange(K)[:, None, None]
    w_ = jnp.arange(win)[None, :, None]
    p_ = jnp.arange(wout // 2)[None, None, :]
    sel_e = (w_ == 2 * p_ + k_).astype(w.dtype)
    sel_o = (w_ == 2 * p_ + 1 + k_).astype(w.dtype)
    cout, cin = w.shape[0], w.shape[1]

    def mk(sel):
        r = jnp.einsum('ochk,kwp->hcwop', w, sel)
        return r.reshape(K * cin * win, cout * (wout // 2)).astype(jnp.bfloat16)

    return mk(sel_e), mk(sel_o)


def kernel(x, conv1_w, conv1_b, conv2_w, conv2_b, fc1_w, fc1_b, fc2_w, fc2_b):
    n = x.shape[0]
    bt = min(B_TILE, n)
    num_tiles = -(-n // bt)
    n_pad = num_tiles * bt

    xr = jnp.transpose(x, (0, 2, 1, 3)).reshape(n, H, CIN * W).astype(jnp.bfloat16)
    if n_pad != n:
        xr = jnp.concatenate(
            [xr, jnp.zeros((n_pad - n,) + xr.shape[1:], xr.dtype)], axis=0)

    r1e, r1o = _banded_eo(conv1_w, W, OW1)
    r2e, r2o = _banded_eo(conv2_w, PW1, OW2)
    b1r = jnp.repeat(conv1_b, PW1).reshape(1, C1 * PW1)
    b2r = jnp.repeat(conv2_b, PW2).reshape(1, C2 * PW2)
    fw1 = jnp.transpose(fc1_w.reshape(H1, C2, PH2, PW2),
                        (2, 1, 3, 0)).reshape(NFEAT, H1).astype(jnp.bfloat16)
    fb1 = fc1_b.reshape(1, H1)
    fw2 = fc2_w.T
    fb2 = fc2_b.reshape(1, NCLS)

    def full(shape):
        zeros = (0,) * len(shape)
        return pl.BlockSpec(shape, lambda g: zeros)

    a2 = pl.pallas_call(
        _conv_kernel,
        out_shape=jax.ShapeDtypeStruct((n_pad, PH2, C2 * PW2), jnp.bfloat16),
        grid=(num_tiles,),
        in_specs=[
            pl.BlockSpec((bt, H, CIN * W), lambda g: (g, 0, 0)),
            full((CIN * K * W, C1 * PW1)), full((CIN * K * W, C1 * PW1)),
            full((1, C1 * PW1)),
            full((C1 * K * PW1, C2 * PW2)), full((C1 * K * PW1, C2 * PW2)),
            full((1, C2 * PW2)),
        ],
        out_specs=pl.BlockSpec((bt, PH2, C2 * PW2), lambda g: (g, 0, 0)),
        scratch_shapes=[
            pltpu.VMEM((bt, OH1, CIN * K * W), jnp.bfloat16),
            pltpu.VMEM((bt, PH1, C1 * PW1), jnp.bfloat16),
            pltpu.VMEM((bt, OH2, C1 * K * PW1), jnp.bfloat16),
        ],
        compiler_params=pltpu.CompilerParams(
            dimension_semantics=("parallel",),
            vmem_limit_bytes=56 * 1024 * 1024),
    )(xr, r1e, r1o, b1r, r2e, r2o, b2r)

    feat = a2.reshape(n_pad, NFEAT)
    bh = min(B_HEAD, n_pad)
    hv_tiles = -(-n_pad // bh)
    n_pad2 = hv_tiles * bh
    if n_pad2 != n_pad:
        feat = jnp.concatenate(
            [feat, jnp.zeros((n_pad2 - n_pad, NFEAT), feat.dtype)], axis=0)

    out = pl.pallas_call(
        _head_kernel,
        out_shape=jax.ShapeDtypeStruct((n_pad2, NCLS), jnp.float32),
        grid=(hv_tiles,),
        in_specs=[
            pl.BlockSpec((bh, NFEAT), lambda g: (g, 0)),
            full((NFEAT, H1)), full((1, H1)),
            full((H1, NCLS)), full((1, NCLS)),
        ],
        out_specs=pl.BlockSpec((bh, NCLS), lambda g: (g, 0)),
        compiler_params=pltpu.CompilerParams(
            dimension_semantics=("parallel",),
            vmem_limit_bytes=56 * 1024 * 1024),
    )(feat, fw1, fb1, fw2, fb2)
    return out[:n]
